# baseline (speedup 1.0000x reference)
import functools

import jax
import jax.numpy as jnp
from jax.experimental import pallas as pl
from jax.experimental.pallas import tpu as pltpu

_TA = 128
_TM = 64


def _pack_mlp(p):
    l, a = p["logic"], p["alpha"]
    wc = jnp.concatenate([l["W1"], a["W1"]], axis=1)
    bc = jnp.concatenate([l["b1"], a["b1"]])[None, :]
    w2 = jnp.zeros((64, 17), jnp.float32)
    w2 = w2.at[0:32, 0:16].set(l["W2"]).at[32:64, 16:17].set(a["W2"])
    b2 = jnp.concatenate([l["b2"], a["b2"]])[None, :]
    return wc, bc, w2, b2


def _rep16(w):
    return jnp.broadcast_to(w, (w.shape[0], 16))


def _pack_l02(p):
    l, a = p["logic"], p["alpha"]
    perm = lambda w: jnp.concatenate([w[32:64], w[0:32]], axis=0)
    wc = jnp.concatenate([l["W1"], a["W1"], perm(l["W1"]), perm(a["W1"])], axis=1)
    bc = jnp.concatenate([l["b1"], a["b1"], l["b1"], a["b1"]])[None, :]
    w_xi, w_a, w_xj, w_b = wc[0:16], wc[16:32], wc[32:48], wc[48:64]
    w2 = jnp.zeros((128, 64), jnp.float32)
    w2 = w2.at[0:32, 0:16].set(l["W2"])
    w2 = w2.at[64:96, 16:32].set(l["W2"])
    w2 = w2.at[32:64, 32:48].set(_rep16(a["W2"]))
    w2 = w2.at[96:128, 48:64].set(_rep16(a["W2"]))
    b2 = jnp.concatenate([l["b2"], l["b2"], _rep16(a["b2"][None])[0],
                          _rep16(a["b2"][None])[0]])[None, :]
    return w_xi, w_a, w_xj, w_b, bc, w2, b2


def _pack_l12(p):
    l, a = p["logic"], p["alpha"]
    wc = jnp.concatenate([l["W1"], a["W1"]], axis=1)
    bc = jnp.concatenate([l["b1"], a["b1"]])[None, :]
    w_ui, w_t, w_uj, w_tp = wc[0:16], wc[16:32], wc[32:48], wc[48:64]
    wq = jnp.concatenate([w_t, w_tp], axis=0)
    w2 = jnp.zeros((64, 32), jnp.float32)
    w2 = w2.at[0:32, 0:16].set(l["W2"])
    w2 = w2.at[32:64, 16:32].set(_rep16(a["W2"]))
    b2 = jnp.concatenate([l["b2"], _rep16(a["b2"][None])[0]])[None, :]
    return w_ui, wq, w_uj, bc, w2, b2


def _dot(x, w):
    return jnp.dot(x, w, preferred_element_type=jnp.float32)


def _sig(x):
    return 0.5 * jnp.tanh(0.5 * x) + 0.5


def _mlp2(x, wc, bc, w2, b2):
    h = jnp.maximum(_dot(x, wc) + bc, 0.0)
    g = _dot(h, w2) + b2
    return _sig(g[:, 0:16]) * _sig(g[:, 16:17])


def _tree_min_max(ex, fa):
    k = ex.shape[1]
    while k > 8:
        h = k // 2
        ex = jnp.maximum(ex[:, :h], ex[:, h:])
        fa = jnp.minimum(fa[:, :h], fa[:, h:])
        k = h
    return jnp.max(ex, axis=1), jnp.min(fa, axis=1)


def _kernel_a(x2_ref, x1_ref, wc0, bc0, w20, b20, wc1, bc1, w21, b21,
              out00_ref, out10_ref, x2t_ref, red_ref, *, nt):
    b = pl.program_id(0)
    i = pl.program_id(1)
    j = pl.program_id(2)
    t = out10_ref.shape[1]
    blk = x2_ref[0]
    w = blk.shape[1]
    x2t_ref[0] = jnp.swapaxes(blk.reshape(t, t, 16), 0, 1)
    ii = jax.lax.broadcasted_iota(jnp.int32, (t, w), 0) + i * t
    jl = jax.lax.broadcasted_iota(jnp.int32, (t, w), 1) // 16 + j * t
    eq = ii == jl
    ex2 = jnp.where(eq, 0.0, blk)
    fa2 = jnp.where(eq, 1.0, blk)
    k = w // 16
    while k > 8:
        h = (k // 2) * 16
        ex2 = jnp.maximum(ex2[:, :h], ex2[:, h:])
        fa2 = jnp.minimum(fa2[:, :h], fa2[:, h:])
        k = k // 2
    ex, fa = _tree_min_max(ex2.reshape(t, k, 16), fa2.reshape(t, k, 16))
    prev = red_ref[...]
    ex = jnp.where(j == 0, ex, jnp.maximum(prev[:, 0:16], ex))
    fa = jnp.where(j == 0, fa, jnp.minimum(prev[:, 16:32], fa))
    red_ref[...] = jnp.concatenate([ex, fa], axis=-1)

    @pl.when(j == nt - 1)
    def _():
        x1i = x1_ref[0, pl.ds(i * t, t), :]
        red = red_ref[...]
        f1 = jnp.concatenate([x1i, red], axis=-1)
        out10_ref[0] = _mlp2(f1, wc1[...], bc1[...], w21[...], b21[...])

    @pl.when(jnp.logical_and(i == 0, j == 0))
    def _():
        x1f = x1_ref[0]
        r1 = jnp.concatenate([jnp.max(x1f, axis=0), jnp.min(x1f, axis=0)])[None, :]
        out00_ref[pl.ds(b, 1), :] = _mlp2(r1, wc0[...], bc0[...], w20[...], b20[...])


def _kernel_m(x2a_ref, x2b_ref, x1i_ref, x1j_ref, u10_ref, u00_ref,
              wxi2, wa2, wxj2, wb2, bc2, w22, b22,
              wui2, wq2, wuj2, bd2, w2d2, b2d2,
              wd1, bd1, w2d1, b2d1,
              wd0, bd0, w2d0, b2d0,
              out2_ref, out1_ref, out0_ref, red_ref, *, nt):
    b = pl.program_id(0)
    i = pl.program_id(1)
    j = pl.program_id(2)
    t = x1i_ref.shape[1]
    m = t * t

    a2 = x2a_ref[0].reshape(m, 16)
    bt2 = x2b_ref[0].reshape(m, 16)
    x1i = x1i_ref[0]
    x1j = x1j_ref[0]

    h2 = _dot(a2, wa2[...]) + _dot(bt2, wb2[...])
    hxi = _dot(x1i, wxi2[...]) + bc2[...]
    hxj = _dot(x1j, wxj2[...])
    h3 = h2.reshape(t, t, 128) + hxi[:, None, :] + hxj[None, :, :]
    h = jnp.maximum(h3, 0.0).reshape(m, 128)
    g = _dot(h, w22[...]) + b22[...]
    s = _sig(g)
    r = s[:, 0:32] * s[:, 32:64]
    tt3 = r[:, 0:16].reshape(t, t, 16)

    ii = jax.lax.broadcasted_iota(jnp.int32, (t, t, 1), 0) + i * t
    jj = jax.lax.broadcasted_iota(jnp.int32, (t, t, 1), 1) + j * t
    eq = ii == jj
    ex, fa = _tree_min_max(jnp.where(eq, 0.0, tt3), jnp.where(eq, 1.0, tt3))
    prev = red_ref[...]
    ex = jnp.where(j == 0, ex, jnp.maximum(prev[:, 0:16], ex))
    fa = jnp.where(j == 0, fa, jnp.minimum(prev[:, 16:32], fa))
    red_ref[...] = jnp.concatenate([ex, fa], axis=-1)

    u1i = u10_ref[0, pl.ds(i * t, t), :]
    u1j = u10_ref[0, pl.ds(j * t, t), :]
    q2 = _dot(r, wq2[...])
    qxi = _dot(u1i, wui2[...]) + bd2[...]
    qxj = _dot(u1j, wuj2[...])
    q3 = q2.reshape(t, t, 64) + qxi[:, None, :] + qxj[None, :, :]
    h1 = jnp.maximum(q3, 0.0).reshape(m, 64)
    g1 = _dot(h1, w2d2[...]) + b2d2[...]
    s1 = _sig(g1)
    out2_ref[0] = (s1[:, 0:16] * s1[:, 16:32]).reshape(t, t, 16)

    @pl.when(j == nt - 1)
    def _():
        red = red_ref[...]
        u00b = jnp.broadcast_to(u00_ref[pl.ds(b, 1), :], (t, 16))
        f1 = jnp.concatenate([u00b, u1i, red], axis=-1)
        out1_ref[0] = _mlp2(f1, wd1[...], bd1[...], w2d1[...], b2d1[...])

    @pl.when(jnp.logical_and(j == nt - 1, i == nt - 1))
    def _():
        u1f = u10_ref[0]
        r1 = jnp.concatenate([jnp.max(u1f, axis=0), jnp.min(u1f, axis=0)])[None, :]
        f0 = jnp.concatenate([u00_ref[pl.ds(b, 1), :], r1], axis=-1)
        out0_ref[pl.ds(b, 1), :] = _mlp2(f0, wd0[...], bd0[...], w2d0[...], b2d0[...])


@jax.jit
def kernel(x1, x2, params):
    bsz, n, c = x1.shape
    t = _TA
    nt = n // t

    p00, p01, p02 = params[0]
    p10, p11, p12 = params[1]
    wa0 = _pack_mlp(p00)
    wa1 = _pack_mlp(p01)
    wm2 = _pack_l02(p02)
    wq2 = _pack_l12(p12)
    wd1 = _pack_mlp(p11)
    wd0 = _pack_mlp(p10)

    x2r = x2.reshape(bsz, n, n * c)

    wfull_a = [pl.BlockSpec(w.shape, functools.partial(lambda nd, b, i, j: (0,) * nd, w.ndim))
               for w in (*wa0, *wa1)]
    out00, out10, x2t = pl.pallas_call(
        functools.partial(_kernel_a, nt=nt),
        grid=(bsz, nt, nt),
        in_specs=[
            pl.BlockSpec((1, t, t * c), lambda b, i, j: (b, i, j)),
            pl.BlockSpec((1, n, c), lambda b, i, j: (b, 0, 0)),
            *wfull_a,
        ],
        out_specs=[
            pl.BlockSpec((bsz, c), lambda b, i, j: (0, 0)),
            pl.BlockSpec((1, t, c), lambda b, i, j: (b, i, 0)),
            pl.BlockSpec((1, t, t, c), lambda b, i, j: (b, j, i, 0)),
        ],
        out_shape=[
            jax.ShapeDtypeStruct((bsz, c), jnp.float32),
            jax.ShapeDtypeStruct((bsz, n, c), jnp.float32),
            jax.ShapeDtypeStruct((bsz, n, n, c), jnp.float32),
        ],
        scratch_shapes=[pltpu.VMEM((t, 2 * c), jnp.float32)],
    )(x2r, x1, *wa0, *wa1)

    tm = _TM
    ntm = n // tm
    weights_m = (*wm2, *wq2, *wd1, *wd0)
    wfull_m = [pl.BlockSpec(w.shape, functools.partial(lambda nd, b, i, j: (0,) * nd, w.ndim))
               for w in weights_m]
    out2, out1, out0 = pl.pallas_call(
        functools.partial(_kernel_m, nt=ntm),
        grid=(bsz, ntm, ntm),
        in_specs=[
            pl.BlockSpec((1, tm, tm, c), lambda b, i, j: (b, i, j, 0)),
            pl.BlockSpec((1, tm, tm, c), lambda b, i, j: (b, i, j, 0)),
            pl.BlockSpec((1, tm, c), lambda b, i, j: (b, i, 0)),
            pl.BlockSpec((1, tm, c), lambda b, i, j: (b, j, 0)),
            pl.BlockSpec((1, n, c), lambda b, i, j: (b, 0, 0)),
            pl.BlockSpec((bsz, c), lambda b, i, j: (0, 0)),
            *wfull_m,
        ],
        out_specs=[
            pl.BlockSpec((1, tm, tm, c), lambda b, i, j: (b, i, j, 0)),
            pl.BlockSpec((1, tm, c), lambda b, i, j: (b, i, 0)),
            pl.BlockSpec((bsz, c), lambda b, i, j: (0, 0)),
        ],
        out_shape=[
            jax.ShapeDtypeStruct((bsz, n, n, c), jnp.float32),
            jax.ShapeDtypeStruct((bsz, n, c), jnp.float32),
            jax.ShapeDtypeStruct((bsz, c), jnp.float32),
        ],
        scratch_shapes=[pltpu.VMEM((tm, 2 * c), jnp.float32)],
    )(x2, x2t, x1, x1, out10, out00, *weights_m)

    return (out0, out1, out2)


# speedup vs baseline: 1.1555x; 1.1555x over previous
import functools

import jax
import jax.numpy as jnp
from jax.experimental import pallas as pl
from jax.experimental.pallas import tpu as pltpu

_TA = 128
_TMI = 128
_TMJ = 64


def _pack_mlp(p):
    l, a = p["logic"], p["alpha"]
    wc = jnp.concatenate([l["W1"], a["W1"]], axis=1)
    bc = jnp.concatenate([l["b1"], a["b1"]])[None, :]
    w2 = jnp.zeros((64, 17), jnp.float32)
    w2 = w2.at[0:32, 0:16].set(l["W2"]).at[32:64, 16:17].set(a["W2"])
    b2 = jnp.concatenate([l["b2"], a["b2"]])[None, :]
    return wc, bc, w2, b2


def _rep16(w):
    return jnp.broadcast_to(w, (w.shape[0], 16))


def _pack_l02(p):
    l, a = p["logic"], p["alpha"]
    perm = lambda w: jnp.concatenate([w[32:64], w[0:32]], axis=0)
    wc = jnp.concatenate([l["W1"], a["W1"], perm(l["W1"]), perm(a["W1"])], axis=1)
    bc = jnp.concatenate([l["b1"], a["b1"], l["b1"], a["b1"]])[None, :]
    w_xi, w_a, w_xj, w_b = wc[0:16], wc[16:32], wc[32:48], wc[48:64]
    w2 = jnp.zeros((128, 64), jnp.float32)
    w2 = w2.at[0:32, 0:16].set(l["W2"])
    w2 = w2.at[64:96, 16:32].set(l["W2"])
    w2 = w2.at[32:64, 32:48].set(_rep16(a["W2"]))
    w2 = w2.at[96:128, 48:64].set(_rep16(a["W2"]))
    b2 = jnp.concatenate([l["b2"], l["b2"], _rep16(a["b2"][None])[0],
                          _rep16(a["b2"][None])[0]])[None, :]
    return w_xi, w_a, w_xj, w_b, bc, 0.5 * w2, 0.5 * b2


def _pack_l12(p):
    l, a = p["logic"], p["alpha"]
    wc = jnp.concatenate([l["W1"], a["W1"]], axis=1)
    bc = jnp.concatenate([l["b1"], a["b1"]])[None, :]
    w_ui, w_t, w_uj, w_tp = wc[0:16], wc[16:32], wc[32:48], wc[48:64]
    wq = 0.25 * jnp.concatenate([w_t, w_tp], axis=0)
    w2 = jnp.zeros((64, 32), jnp.float32)
    w2 = w2.at[0:32, 0:16].set(l["W2"])
    w2 = w2.at[32:64, 16:32].set(_rep16(a["W2"]))
    b2 = jnp.concatenate([l["b2"], _rep16(a["b2"][None])[0]])[None, :]
    return w_ui, wq, w_uj, bc, 0.5 * w2, 0.5 * b2


def _dot(x, w):
    return jnp.dot(x, w, preferred_element_type=jnp.float32)


def _sig(x):
    return 0.5 * jnp.tanh(0.5 * x) + 0.5


def _mlp2(x, wc, bc, w2, b2):
    h = jnp.maximum(_dot(x, wc) + bc, 0.0)
    g = _dot(h, w2) + b2
    return _sig(g[:, 0:16]) * _sig(g[:, 16:17])


def _tree_min_max(ex, fa):
    k = ex.shape[1]
    while k > 8:
        h = k // 2
        ex = jnp.maximum(ex[:, :h], ex[:, h:])
        fa = jnp.minimum(fa[:, :h], fa[:, h:])
        k = h
    return jnp.max(ex, axis=1), jnp.min(fa, axis=1)


def _kernel_a(x2_ref, x1_ref, wc0, bc0, w20, b20, wc1, bc1, w21, b21,
              out00_ref, out10_ref, red_ref, *, nt):
    b = pl.program_id(0)
    i = pl.program_id(1)
    j = pl.program_id(2)
    t = out10_ref.shape[1]
    blk = x2_ref[0]
    w = blk.shape[1]
    ii = jax.lax.broadcasted_iota(jnp.int32, (t, w), 0) + i * t
    jl = jax.lax.broadcasted_iota(jnp.int32, (t, w), 1) // 16 + j * t
    eq = ii == jl
    ex2 = jnp.where(eq, 0.0, blk)
    fa2 = jnp.where(eq, 1.0, blk)
    k = w // 16
    while k > 8:
        h = (k // 2) * 16
        ex2 = jnp.maximum(ex2[:, :h], ex2[:, h:])
        fa2 = jnp.minimum(fa2[:, :h], fa2[:, h:])
        k = k // 2
    ex, fa = _tree_min_max(ex2.reshape(t, k, 16), fa2.reshape(t, k, 16))
    prev = red_ref[...]
    ex = jnp.where(j == 0, ex, jnp.maximum(prev[:, 0:16], ex))
    fa = jnp.where(j == 0, fa, jnp.minimum(prev[:, 16:32], fa))
    red_ref[...] = jnp.concatenate([ex, fa], axis=-1)

    @pl.when(j == nt - 1)
    def _():
        x1i = x1_ref[0, pl.ds(i * t, t), :]
        red = red_ref[...]
        f1 = jnp.concatenate([x1i, red], axis=-1)
        out10_ref[0] = _mlp2(f1, wc1[...], bc1[...], w21[...], b21[...])

    @pl.when(jnp.logical_and(i == 0, j == 0))
    def _():
        x1f = x1_ref[0]
        r1 = jnp.concatenate([jnp.max(x1f, axis=0), jnp.min(x1f, axis=0)])[None, :]
        out00_ref[pl.ds(b, 1), :] = _mlp2(r1, wc0[...], bc0[...], w20[...], b20[...])


def _kernel_m(x2a_ref, x2b_ref, x1i_ref, x1j_ref, u10_ref, u00_ref,
              wxi2, wa2, wxj2, wb2, bc2, w22, b22,
              wui2, wq2, wuj2, bd2, w2d2, b2d2,
              wd1, bd1, w2d1, b2d1,
              wd0, bd0, w2d0, b2d0,
              out2_ref, out1_ref, out0_ref, red_ref, *, nti, ntj):
    b = pl.program_id(0)
    i = pl.program_id(1)
    j = pl.program_id(2)
    ti = x1i_ref.shape[1]
    tj = x1j_ref.shape[1]
    m = ti * tj

    a2 = x2a_ref[0].reshape(m, 16)
    bt2 = x2b_ref[0].reshape(m, 16)
    x1i = x1i_ref[0]
    x1j = x1j_ref[0]

    h2 = _dot(a2, wa2[...]) + _dot(bt2, wb2[...])
    hxi = _dot(x1i, wxi2[...]) + bc2[...]
    hxj = _dot(x1j, wxj2[...])
    h3 = h2.reshape(ti, tj, 128) + hxi[:, None, :] + hxj[None, :, :]
    h = jnp.maximum(h3, 0.0).reshape(m, 128)
    g = jnp.tanh(_dot(h, w22[...]) + b22[...])
    r = (1.0 + g[:, 0:32]) * (1.0 + g[:, 32:64])
    tt3 = r[:, 0:16].reshape(ti, tj, 16)

    ii = jax.lax.broadcasted_iota(jnp.int32, (ti, tj, 1), 0) + i * ti
    jj = jax.lax.broadcasted_iota(jnp.int32, (ti, tj, 1), 1) + j * tj
    eq = ii == jj
    ex, fa = _tree_min_max(jnp.where(eq, 0.0, tt3), jnp.where(eq, 4.0, tt3))
    prev = red_ref[...]
    ex = jnp.where(j == 0, ex, jnp.maximum(prev[:, 0:16], ex))
    fa = jnp.where(j == 0, fa, jnp.minimum(prev[:, 16:32], fa))
    red_ref[...] = jnp.concatenate([ex, fa], axis=-1)

    u1i = u10_ref[0, pl.ds(i * ti, ti), :]
    u1j = u10_ref[0, pl.ds(j * tj, tj), :]
    q2 = _dot(r, wq2[...])
    qxi = _dot(u1i, wui2[...]) + bd2[...]
    qxj = _dot(u1j, wuj2[...])
    q3 = q2.reshape(ti, tj, 64) + qxi[:, None, :] + qxj[None, :, :]
    h1 = jnp.maximum(q3, 0.0).reshape(m, 64)
    g1 = jnp.tanh(_dot(h1, w2d2[...]) + b2d2[...])
    o = (0.25 + 0.25 * g1[:, 0:16]) * (1.0 + g1[:, 16:32])
    out2_ref[0] = o.reshape(ti, tj, 16)

    @pl.when(j == ntj - 1)
    def _():
        red = red_ref[...]
        u00b = jnp.broadcast_to(u00_ref[pl.ds(b, 1), :], (ti, 16))
        f1 = jnp.concatenate([u00b, u1i, red], axis=-1)
        out1_ref[0] = _mlp2(f1, wd1[...], bd1[...], w2d1[...], b2d1[...])

    @pl.when(jnp.logical_and(j == ntj - 1, i == nti - 1))
    def _():
        u1f = u10_ref[0]
        r1 = jnp.concatenate([jnp.max(u1f, axis=0), jnp.min(u1f, axis=0)])[None, :]
        f0 = jnp.concatenate([u00_ref[pl.ds(b, 1), :], r1], axis=-1)
        out0_ref[pl.ds(b, 1), :] = _mlp2(f0, wd0[...], bd0[...], w2d0[...], b2d0[...])


@jax.jit
def kernel(x1, x2, params):
    bsz, n, c = x1.shape
    t = _TA
    nt = n // t

    p00, p01, p02 = params[0]
    p10, p11, p12 = params[1]
    wa0 = _pack_mlp(p00)
    wa1 = _pack_mlp(p01)
    wm2 = _pack_l02(p02)
    wq2 = _pack_l12(p12)
    wd1 = _pack_mlp(p11)
    wd1 = (wd1[0].at[32:64].multiply(0.25), *wd1[1:])
    wd0 = _pack_mlp(p10)

    x2t = jnp.swapaxes(x2, 1, 2)
    x2r = x2.reshape(bsz, n, n * c)

    wfull_a = [pl.BlockSpec(w.shape, functools.partial(lambda nd, b, i, j: (0,) * nd, w.ndim))
               for w in (*wa0, *wa1)]
    out00, out10 = pl.pallas_call(
        functools.partial(_kernel_a, nt=nt),
        grid=(bsz, nt, nt),
        in_specs=[
            pl.BlockSpec((1, t, t * c), lambda b, i, j: (b, i, j)),
            pl.BlockSpec((1, n, c), lambda b, i, j: (b, 0, 0)),
            *wfull_a,
        ],
        out_specs=[
            pl.BlockSpec((bsz, c), lambda b, i, j: (0, 0)),
            pl.BlockSpec((1, t, c), lambda b, i, j: (b, i, 0)),
        ],
        out_shape=[
            jax.ShapeDtypeStruct((bsz, c), jnp.float32),
            jax.ShapeDtypeStruct((bsz, n, c), jnp.float32),
        ],
        scratch_shapes=[pltpu.VMEM((t, 2 * c), jnp.float32)],
    )(x2r, x1, *wa0, *wa1)

    ti, tj = _TMI, _TMJ
    nti, ntj = n // ti, n // tj
    weights_m = (*wm2, *wq2, *wd1, *wd0)
    wfull_m = [pl.BlockSpec(w.shape, functools.partial(lambda nd, b, i, j: (0,) * nd, w.ndim))
               for w in weights_m]
    out2, out1, out0 = pl.pallas_call(
        functools.partial(_kernel_m, nti=nti, ntj=ntj),
        grid=(bsz, nti, ntj),
        in_specs=[
            pl.BlockSpec((1, ti, tj, c), lambda b, i, j: (b, i, j, 0)),
            pl.BlockSpec((1, ti, tj, c), lambda b, i, j: (b, i, j, 0)),
            pl.BlockSpec((1, ti, c), lambda b, i, j: (b, i, 0)),
            pl.BlockSpec((1, tj, c), lambda b, i, j: (b, j, 0)),
            pl.BlockSpec((1, n, c), lambda b, i, j: (b, 0, 0)),
            pl.BlockSpec((bsz, c), lambda b, i, j: (0, 0)),
            *wfull_m,
        ],
        out_specs=[
            pl.BlockSpec((1, ti, tj, c), lambda b, i, j: (b, i, j, 0)),
            pl.BlockSpec((1, ti, c), lambda b, i, j: (b, i, 0)),
            pl.BlockSpec((bsz, c), lambda b, i, j: (0, 0)),
        ],
        out_shape=[
            jax.ShapeDtypeStruct((bsz, n, n, c), jnp.float32),
            jax.ShapeDtypeStruct((bsz, n, c), jnp.float32),
            jax.ShapeDtypeStruct((bsz, c), jnp.float32),
        ],
        scratch_shapes=[pltpu.VMEM((ti, 2 * c), jnp.float32)],
    )(x2, x2t, x1, x1, out10, out00, *weights_m)

    return (out0, out1, out2)
